# baseline (speedup 1.0000x reference)
import dataclasses
import functools

import jax
import jax.numpy as jnp
from jax import lax
from jax.experimental import pallas as pl
from jax.experimental.pallas import tpu as pltpu
from jax.experimental.pallas import tpu_sc as plsc

N = 100000
M = 32
B = 4
D = 3

NUM_TILES = 32
CHUNK = 64
NODES_PER_TILE = 3136
NCHUNKS = NODES_PER_TILE // CHUNK
IDX_ROWS = CHUNK * M // 128
WROW = 136


def _sc_body(fs16_hbm, idx_hbm, w_hbm, out_hbm,
             idx0, idx1, g0, g1, w0, w1, accbuf, resbuf, sem0, sem1):
    cid = lax.axis_index("c")
    sid = lax.axis_index("s")
    wid = cid * 16 + sid
    tile_base = jnp.minimum(wid * NODES_PER_TILE, N - NODES_PER_TILE)

    lane = jnp.arange(16, dtype=jnp.int32)
    patt = jnp.where((lane & 3) < D, (lane & 3) * 33, 108)
    patts = tuple(jnp.where((lane & 3) < D, patt + j, patt) for j in range(8))

    slots = ((idx0, g0, w0, sem0), (idx1, g1, w1, sem1))

    zero16 = jnp.zeros((16,), jnp.float32)

    @pl.loop(0, CHUNK)
    def _zrow(n):
        for wbuf in (w0, w1):
            wbuf[n, pl.ds(96, 16)] = zero16
            wbuf[n, pl.ds(112, 16)] = zero16
            wbuf[n, pl.ds(120, 16)] = zero16

    def fire(i, slot):
        idxb, gb, wb, sem = slots[slot]
        base = pl.multiple_of(tile_base + i * CHUNK, 32)
        pltpu.sync_copy(idx_hbm.at[pl.ds(base * M // 128, IDX_ROWS)], idxb)
        for j in range(IDX_ROWS):
            pltpu.async_copy(fs16_hbm.at[idxb.at[j]],
                             gb.at[pl.ds(j * 128, 128)], sem)
        pltpu.async_copy(w_hbm.at[pl.ds(base, CHUNK)],
                         wb.at[pl.ds(0, CHUNK), pl.ds(0, D * M)], sem)

    def drain(slot):
        idxb, gb, wb, sem = slots[slot]
        pltpu.make_async_copy(fs16_hbm.at[pl.ds(0, CHUNK * M)], gb, sem).wait()
        pltpu.make_async_copy(
            w_hbm.at[pl.ds(0, CHUNK)],
            wb.at[pl.ds(0, CHUNK), pl.ds(0, D * M)], sem).wait()

    def compute(i, slot):
        _, gb, wb, _ = slots[slot]

        @pl.loop(0, CHUNK)
        def _skew(n):
            row = wb.at[n]
            v2a = wb[n, pl.ds(2 * M, 16)]
            v2b = wb[n, pl.ds(2 * M + 16, 16)]
            plsc.store_scatter(row, [lane + 66], v2a)
            plsc.store_scatter(row, [lane + 82], v2b)
            v1a = wb[n, pl.ds(M, 16)]
            v1b = wb[n, pl.ds(M + 16, 16)]
            plsc.store_scatter(row, [lane + 33], v1a)
            plsc.store_scatter(row, [lane + 49], v1b)

        @pl.loop(0, CHUNK)
        def _node(n):
            acc0 = jnp.zeros((16,), jnp.float32)
            acc1 = jnp.zeros((16,), jnp.float32)
            acc2 = jnp.zeros((16,), jnp.float32)
            acc3 = jnp.zeros((16,), jnp.float32)
            accs = [acc0, acc1, acc2, acc3]
            for m in range(M):
                wv = plsc.load_gather(
                    wb.at[n, pl.ds(m & ~7, 112)], [patts[m & 7]])
                gv = gb[n * M + m]
                accs[m & 3] = accs[m & 3] + wv * gv
            acc = (accs[0] + accs[1]) + (accs[2] + accs[3])
            accbuf[pl.ds(n * 16, 16)] = acc

        @pl.loop(0, CHUNK // 16)
        def _fold(g):
            rows = (g * 16 + lane) * 16
            for b in range(B):
                s0 = plsc.load_gather(accbuf, [rows + (4 * b + 0)])
                s1 = plsc.load_gather(accbuf, [rows + (4 * b + 1)])
                s2 = plsc.load_gather(accbuf, [rows + (4 * b + 2)])
                s3 = plsc.load_gather(accbuf, [rows + (4 * b + 3)])
                resbuf[pl.ds(b * NODES_PER_TILE + i * CHUNK + g * 16, 16)] = (
                    (s0 + s1) + (s2 + s3))

    fire(0, 0)

    @pl.loop(0, NCHUNKS - 1, step=2)
    def _pair(g):
        fire(g + 1, 1)
        drain(0)
        compute(g, 0)
        fire(g + 2, 0)
        drain(1)
        compute(g + 1, 1)

    drain(0)
    compute(NCHUNKS - 1, 0)

    for b in range(B):
        pltpu.sync_copy(
            resbuf.at[pl.ds(b * NODES_PER_TILE, NODES_PER_TILE)],
            out_hbm.at[pl.ds(b * N + tile_base, NODES_PER_TILE)])


@jax.jit
def _rbffd_divergence_sc(fs16, idx2d, w_flat):
    mesh = plsc.VectorSubcoreMesh(core_axis_name="c", subcore_axis_name="s")
    cp = pltpu.CompilerParams()
    if "needs_layout_passes" in pltpu.CompilerParams.__dataclass_fields__:
        cp = dataclasses.replace(cp, needs_layout_passes=False)
    if "use_tc_tiling_on_sc" in pltpu.CompilerParams.__dataclass_fields__:
        cp = dataclasses.replace(cp, use_tc_tiling_on_sc=False)
    run = pl.kernel(
        _sc_body,
        out_type=jax.ShapeDtypeStruct((B * N,), jnp.float32),
        mesh=mesh,
        scratch_types=[
            pltpu.VMEM((IDX_ROWS, 128), jnp.int32),
            pltpu.VMEM((IDX_ROWS, 128), jnp.int32),
            pltpu.VMEM((CHUNK * M, 16), jnp.float32),
            pltpu.VMEM((CHUNK * M, 16), jnp.float32),
            pltpu.VMEM((CHUNK, WROW), jnp.float32),
            pltpu.VMEM((CHUNK, WROW), jnp.float32),
            pltpu.VMEM((CHUNK * 16,), jnp.float32),
            pltpu.VMEM((B * NODES_PER_TILE,), jnp.float32),
            pltpu.SemaphoreType.DMA,
            pltpu.SemaphoreType.DMA,
        ],
        compiler_params=cp,
    )
    return run(fs16, idx2d, w_flat)


def kernel(fs, stencil_indices, weights):
    fs = jnp.asarray(fs, jnp.float32)
    fs16 = jnp.pad(jnp.transpose(fs, (1, 0, 2)),
                   ((0, 0), (0, 0), (0, 1))).reshape(N, 4 * B)
    idx2d = stencil_indices.reshape(N * M // 128, 128)
    w2d = jnp.asarray(weights, jnp.float32).reshape(N, D * M)
    out_flat = _rbffd_divergence_sc(fs16, idx2d, w2d)
    return out_flat.reshape(B, N)


# speedup vs baseline: 1.1134x; 1.1134x over previous
import dataclasses
import functools

import jax
import jax.numpy as jnp
from jax import lax
from jax.experimental import pallas as pl
from jax.experimental.pallas import tpu as pltpu
from jax.experimental.pallas import tpu_sc as plsc

N = 100000
M = 32
B = 4
D = 3

NUM_TILES = 32
CHUNK = 64
NODES_PER_TILE = 3136
NCHUNKS = NODES_PER_TILE // CHUNK
IDX_ROWS = CHUNK * M // 128
WROW = 136


def _sc_body(fs16_hbm, idx_hbm, w_hbm, out_hbm,
             idx0, idx1, g0, g1, w0, w1, wsk, accbuf, resbuf, sem0, sem1):
    cid = lax.axis_index("c")
    sid = lax.axis_index("s")
    wid = cid * 16 + sid
    tile_base = jnp.minimum(wid * NODES_PER_TILE, N - NODES_PER_TILE)

    lane = jnp.arange(16, dtype=jnp.int32)
    patt = jnp.where((lane & 3) < D, (lane & 3) * 33, 108)
    patts = tuple(jnp.where((lane & 3) < D, patt + j, patt) for j in range(8))

    slots = ((idx0, g0, w0, sem0), (idx1, g1, w1, sem1))

    zero16 = jnp.zeros((16,), jnp.float32)

    @pl.loop(0, CHUNK)
    def _zrow(n):
        wsk[n, pl.ds(96, 16)] = zero16
        wsk[n, pl.ds(112, 16)] = zero16
        wsk[n, pl.ds(120, 16)] = zero16

    def fire(i, slot):
        idxb, gb, wb, sem = slots[slot]
        base = pl.multiple_of(tile_base + i * CHUNK, 32)
        pltpu.sync_copy(idx_hbm.at[pl.ds(base * M // 128, IDX_ROWS)], idxb)
        for j in range(IDX_ROWS):
            pltpu.async_copy(fs16_hbm.at[idxb.at[j]],
                             gb.at[pl.ds(j * 128, 128)], sem)
        pltpu.async_copy(w_hbm.at[pl.ds(base * D * M, CHUNK * D * M)], wb, sem)

    def drain(slot):
        idxb, gb, wb, sem = slots[slot]
        pltpu.make_async_copy(fs16_hbm.at[pl.ds(0, CHUNK * M)], gb, sem).wait()
        pltpu.make_async_copy(
            w_hbm.at[pl.ds(0, CHUNK * D * M)], wb, sem).wait()

    def compute(i, slot):
        _, gb, wb, _ = slots[slot]

        @pl.loop(0, CHUNK)
        def _skew(n):
            rowbase = n * (D * M)
            row = wsk.at[n]
            v0a = wb[pl.ds(rowbase, 16)]
            v0b = wb[pl.ds(rowbase + 16, 16)]
            wsk[n, pl.ds(0, 16)] = v0a
            wsk[n, pl.ds(16, 16)] = v0b
            v1a = wb[pl.ds(rowbase + 32, 16)]
            v1b = wb[pl.ds(rowbase + 48, 16)]
            plsc.store_scatter(row, [lane + 33], v1a)
            plsc.store_scatter(row, [lane + 49], v1b)
            v2a = wb[pl.ds(rowbase + 64, 16)]
            v2b = wb[pl.ds(rowbase + 80, 16)]
            plsc.store_scatter(row, [lane + 66], v2a)
            plsc.store_scatter(row, [lane + 82], v2b)

        @pl.loop(0, CHUNK)
        def _node(n):
            acc0 = jnp.zeros((16,), jnp.float32)
            acc1 = jnp.zeros((16,), jnp.float32)
            acc2 = jnp.zeros((16,), jnp.float32)
            acc3 = jnp.zeros((16,), jnp.float32)
            accs = [acc0, acc1, acc2, acc3]
            for m in range(M):
                wv = plsc.load_gather(
                    wsk.at[n, pl.ds(m & ~7, 112)], [patts[m & 7]])
                gv = gb[n * M + m]
                accs[m & 3] = accs[m & 3] + wv * gv
            acc = (accs[0] + accs[1]) + (accs[2] + accs[3])
            accbuf[pl.ds(n * 16, 16)] = acc

        @pl.loop(0, CHUNK // 16)
        def _fold(g):
            rows = (g * 16 + lane) * 16
            for b in range(B):
                s0 = plsc.load_gather(accbuf, [rows + (4 * b + 0)])
                s1 = plsc.load_gather(accbuf, [rows + (4 * b + 1)])
                s2 = plsc.load_gather(accbuf, [rows + (4 * b + 2)])
                s3 = plsc.load_gather(accbuf, [rows + (4 * b + 3)])
                resbuf[pl.ds(b * NODES_PER_TILE + i * CHUNK + g * 16, 16)] = (
                    (s0 + s1) + (s2 + s3))

    fire(0, 0)

    @pl.loop(0, NCHUNKS - 1, step=2)
    def _pair(g):
        fire(g + 1, 1)
        drain(0)
        compute(g, 0)
        fire(g + 2, 0)
        drain(1)
        compute(g + 1, 1)

    drain(0)
    compute(NCHUNKS - 1, 0)

    for b in range(B):
        pltpu.sync_copy(
            resbuf.at[pl.ds(b * NODES_PER_TILE, NODES_PER_TILE)],
            out_hbm.at[pl.ds(b * N + tile_base, NODES_PER_TILE)])


@jax.jit
def _rbffd_divergence_sc(fs16, idx2d, w_flat):
    mesh = plsc.VectorSubcoreMesh(core_axis_name="c", subcore_axis_name="s")
    cp = pltpu.CompilerParams()
    if "needs_layout_passes" in pltpu.CompilerParams.__dataclass_fields__:
        cp = dataclasses.replace(cp, needs_layout_passes=False)
    if "use_tc_tiling_on_sc" in pltpu.CompilerParams.__dataclass_fields__:
        cp = dataclasses.replace(cp, use_tc_tiling_on_sc=False)
    run = pl.kernel(
        _sc_body,
        out_type=jax.ShapeDtypeStruct((B * N,), jnp.float32),
        mesh=mesh,
        scratch_types=[
            pltpu.VMEM((IDX_ROWS, 128), jnp.int32),
            pltpu.VMEM((IDX_ROWS, 128), jnp.int32),
            pltpu.VMEM((CHUNK * M, 16), jnp.float32),
            pltpu.VMEM((CHUNK * M, 16), jnp.float32),
            pltpu.VMEM((CHUNK * D * M,), jnp.float32),
            pltpu.VMEM((CHUNK * D * M,), jnp.float32),
            pltpu.VMEM((CHUNK, WROW), jnp.float32),
            pltpu.VMEM((CHUNK * 16,), jnp.float32),
            pltpu.VMEM((B * NODES_PER_TILE,), jnp.float32),
            pltpu.SemaphoreType.DMA,
            pltpu.SemaphoreType.DMA,
        ],
        compiler_params=cp,
    )
    return run(fs16, idx2d, w_flat)


def kernel(fs, stencil_indices, weights):
    fs = jnp.asarray(fs, jnp.float32)
    fs16 = jnp.pad(jnp.transpose(fs, (1, 0, 2)),
                   ((0, 0), (0, 0), (0, 1))).reshape(N, 4 * B)
    idx2d = stencil_indices.reshape(N * M // 128, 128)
    w_flat = jnp.asarray(weights, jnp.float32).reshape(-1)
    out_flat = _rbffd_divergence_sc(fs16, idx2d, w_flat)
    return out_flat.reshape(B, N)
